# baseline (speedup 1.0000x reference)
import functools

import jax
import jax.numpy as jnp
from jax import lax
from jax.experimental import pallas as pl
from jax.experimental.pallas import tpu as pltpu
from jax.experimental.pallas import tpu_sc as plsc

_N_NODES = 10000
_N_PAD = 10240
_N_EDGES = 320000
_DIM = 128
_NC = 2
_NS = 16
_NW = _NC * _NS
_K = 80
_CPW = 128
_E_PAD = _NW * _CPW * _K
_RPT = _N_PAD // _NS
_ZR = _K
_NB = 4


def _sc_aggregate(x, src, dst):
    mesh = plsc.VectorSubcoreMesh(core_axis_name="c", subcore_axis_name="s")

    @functools.partial(
        pl.kernel,
        out_type=jax.ShapeDtypeStruct((_NC, _N_PAD, _DIM), jnp.float32),
        mesh=mesh,
        scratch_types=[
            pltpu.VMEM_SHARED((_N_PAD, _DIM), jnp.float32),
            [pltpu.VMEM((_K, _DIM), jnp.float32)] * _NB,
            [pltpu.VMEM((_K,), jnp.int32)] * (2 * _NB),
            [pltpu.VMEM((_K,), jnp.int32)] * 2,
            [pltpu.SemaphoreType.DMA] * _NB,
            [pltpu.SemaphoreType.DMA] * (2 * _NB),
            [pltpu.SemaphoreType.DMA] * 2,
        ],
    )
    def k(x_hbm, src_hbm, dst_hbm, out_hbm, accum, ring, sidx, didx,
          gsem, ssem, dsem):
        c = lax.axis_index("c")
        s = lax.axis_index("s")
        w = s * _NC + c
        e0 = w * _CPW * _K

        def zero_row(r, carry):
            for j in range(_DIM // 16):
                ring[0][r, pl.ds(j * 16, 16)] = jnp.zeros((16,), jnp.float32)
            return carry

        lax.fori_loop(0, _ZR, zero_row, 0)
        row0 = s * _RPT
        for j in range(_RPT // _ZR):
            pltpu.sync_copy(ring[0], accum.at[pl.ds(row0 + j * _ZR, _ZR)])
        plsc.subcore_barrier()

        def sload(j, b):
            pltpu.async_copy(src_hbm.at[pl.ds(e0 + j * _K, _K)],
                             sidx[b], ssem[b])

        def swait(j, b):
            pltpu.make_async_copy(src_hbm.at[pl.ds(e0 + j * _K, _K)],
                                  sidx[b], ssem[b]).wait()

        def dload(j, p):
            pltpu.async_copy(dst_hbm.at[pl.ds(e0 + j * _K, _K)],
                             didx[p], dsem[p])

        def dwait(j, p):
            pltpu.make_async_copy(dst_hbm.at[pl.ds(e0 + j * _K, _K)],
                                  didx[p], dsem[p]).wait()

        def gstart(sb, rb):
            pltpu.async_copy(x_hbm.at[sidx[sb]], ring[rb], gsem[rb])

        def gwait(sb, rb):
            pltpu.make_async_copy(
                x_hbm.at[sidx[sb]], ring[rb], gsem[rb]).wait()

        def scatter(rb, p):
            pltpu.sync_copy(ring[rb], accum.at[didx[p]], add=True)

        for b in range(2 * _NB):
            sload(b, b)
        dload(0, 0)
        for b in range(_NB):
            swait(b, b)
            gstart(b, b)

        def octet(i, carry):
            j0 = 8 * i
            for b in range(2 * _NB):
                j = j0 + b
                dload(j + 1, (b + 1) % 2)
                gwait(b, b % _NB)
                dwait(j, b % 2)
                scatter(b % _NB, b % 2)
                swait(j + _NB, (b + _NB) % 8)
                gstart((b + _NB) % 8, b % _NB)
                sload(j + 2 * _NB, b)
            return carry

        lax.fori_loop(0, _CPW // (2 * _NB) - 1, octet, 0)

        for b in range(2 * _NB):
            j = _CPW - 2 * _NB + b
            if b + 1 < 2 * _NB:
                dload(j + 1, (b + 1) % 2)
            gwait(b, b % _NB)
            dwait(j, b % 2)
            scatter(b % _NB, b % 2)
            if b < _NB:
                swait(j + _NB, (b + _NB) % 8)
                gstart((b + _NB) % 8, b % _NB)
        plsc.subcore_barrier()

        for j in range(_RPT // _ZR):
            r = row0 + j * _ZR
            pltpu.sync_copy(accum.at[pl.ds(r, _ZR)], ring[0])
            pltpu.sync_copy(ring[0], out_hbm.at[c].at[pl.ds(r, _ZR)])

    return k(x, src, dst)


def _mm_relu(partials, W):
    blk = 1024

    def body(p0_ref, p1_ref, w_ref, o_ref):
        ssum = p0_ref[...] + p1_ref[...]
        o_ref[...] = jnp.maximum(
            jnp.dot(ssum, w_ref[...], preferred_element_type=jnp.float32),
            0.0)

    return pl.pallas_call(
        body,
        grid=(_N_PAD // blk,),
        in_specs=[
            pl.BlockSpec((blk, _DIM), lambda i: (i, 0)),
            pl.BlockSpec((blk, _DIM), lambda i: (i, 0)),
            pl.BlockSpec((_DIM, _DIM), lambda i: (0, 0)),
        ],
        out_specs=pl.BlockSpec((blk, _DIM), lambda i: (i, 0)),
        out_shape=jax.ShapeDtypeStruct((_N_NODES, _DIM), jnp.float32),
    )(partials[0], partials[1], W)


def kernel(x, edge_index, W):
    src = edge_index[1].astype(jnp.int32)
    dst = edge_index[0].astype(jnp.int32)
    npad = _E_PAD - _N_EDGES
    src_p = jnp.concatenate([src, jnp.zeros((npad,), jnp.int32)])
    dst_p = jnp.concatenate([dst, jnp.full((npad,), _N_NODES, jnp.int32)])
    partials = _sc_aggregate(x, src_p, dst_p)
    return _mm_relu(partials, W)
